# baseline (speedup 1.0000x reference)
import jax
import jax.numpy as jnp
from jax import lax
from jax.experimental import pallas as pl
from jax.experimental.pallas import tpu as pltpu
from jax.experimental.pallas import tpu_sc as plsc

_NC, _NS = 2, 16
_NW = _NC * _NS
_BB = 128

_TCOLS = 32768


def _tbl_body(in_ref, out_ref):
    x = in_ref[...]
    xt = x.T
    n = xt.shape[0]
    out_ref[...] = jnp.concatenate([xt[:n // 2], xt[n // 2:]], axis=1)


def _convert_table(wemb):
    v, h = wemb.shape
    pairs = 128 // h
    grid = (v + _TCOLS - 1) // _TCOLS
    rows = grid * (_TCOLS // pairs)
    out = pl.pallas_call(
        _tbl_body,
        grid=(grid,),
        in_specs=[pl.BlockSpec((h, _TCOLS), lambda i: (0, i))],
        out_specs=pl.BlockSpec((_TCOLS // pairs, 128), lambda i: (i, 0)),
        out_shape=jax.ShapeDtypeStruct((rows, 128), jnp.float32),
    )(wemb.T)
    return out.reshape(rows * pairs, h)


def _remap_indices(v):
    half = _TCOLS // 2
    return (((v // _TCOLS) * _TCOLS) + ((v % half) * 2) + ((v // half) % 2))


def _gather_body(idxT_hbm, wemb_hbm, out_hbm, idx_all, rows_a, rows_b,
                 gsem, osem):
    c = lax.axis_index("c")
    s = lax.axis_index("s")
    w = s * _NC + c
    seq = idxT_hbm.shape[0]
    b0 = w * _BB

    pltpu.sync_copy(idxT_hbm.at[:, pl.ds(b0, _BB)], idx_all)
    pltpu.async_copy(wemb_hbm.at[idx_all.at[0]], rows_a, gsem)

    def halfstep(t, cur, nxt):
        pltpu.make_async_copy(wemb_hbm.at[idx_all.at[t]], cur, gsem).wait()

        @pl.when(t >= 1)
        def _drain_prev():
            pltpu.make_async_copy(nxt, out_hbm.at[t - 1, w], osem).wait()

        @pl.when(t + 1 < seq)
        def _start_next():
            pltpu.async_copy(wemb_hbm.at[idx_all.at[t + 1]], nxt, gsem)

        pltpu.async_copy(cur, out_hbm.at[t, w], osem)

    def pair(i, carry):
        t = i * 2
        halfstep(t, rows_a, rows_b)
        halfstep(t + 1, rows_b, rows_a)
        return carry

    lax.fori_loop(0, seq // 2, pair, 0)
    pltpu.make_async_copy(rows_b, out_hbm.at[seq - 1, w], osem).wait()


def _ln_body(x_ref, p_ref, g_ref, b_ref, o_ref):
    x = x_ref[...]
    t = pl.program_id(0)
    p = p_ref[pl.ds(t, 1)]
    g = g_ref[...]
    b = b_ref[...]

    def ln(e):
        u = jnp.mean(e, axis=-1, keepdims=True)
        s2 = jnp.mean((e - u) ** 2, axis=-1, keepdims=True)
        return g * ((e - u) * lax.rsqrt(s2 + 1e-12)) + b

    n0 = ln(x[:, :64] + p).reshape(8, 64, 64)
    n1 = ln(x[:, 64:] + p).reshape(8, 64, 64)
    per_bo = []
    for i in range(8):
        ev = n0[i].T
        od = n1[i].T
        inter = jnp.stack([ev, od], axis=-1).reshape(64, 128)
        per_bo.append(inter.reshape(8, 8, 128))
    ob = jnp.stack(per_bo, axis=1)
    o_ref[...] = ob[None]


def _tc_epilogue(emb4, posemb, gamma, beta):
    seq, nbo, bb, h = emb4.shape
    pairs = emb4.reshape(seq * nbo * (bb // 2), 2 * h)
    out5 = pl.pallas_call(
        _ln_body,
        grid=(seq, nbo // 8),
        in_specs=[
            pl.BlockSpec((512, 128), lambda t, g: (t * 4 + g, 0)),
            pl.BlockSpec((seq, h), lambda t, g: (0, 0)),
            pl.BlockSpec((1, h), lambda t, g: (0, 0)),
            pl.BlockSpec((1, h), lambda t, g: (0, 0)),
        ],
        out_specs=pl.BlockSpec((1, 8, 8, 8, 128),
                               lambda t, g: (t, 0, g, 0, 0)),
        out_shape=jax.ShapeDtypeStruct((seq, 8, nbo, 8, bb), jnp.float32),
    )(pairs, posemb, gamma.reshape(1, h), beta.reshape(1, h))
    return out5


def kernel(inputni, wemb, posemb, gamma, beta):
    batch, seq = inputni.shape
    h = wemb.shape[1]
    idxT = _remap_indices(inputni.T.astype(jnp.int32))
    wemb_lin = _convert_table(wemb)
    mesh = plsc.VectorSubcoreMesh(core_axis_name="c", subcore_axis_name="s")
    emb4 = pl.kernel(
        _gather_body,
        out_type=jax.ShapeDtypeStruct((seq, batch // _BB, _BB, h),
                                      jnp.float32),
        mesh=mesh,
        scratch_types=[
            pltpu.VMEM((seq, _BB), jnp.int32),
            pltpu.VMEM((_BB, h), jnp.float32),
            pltpu.VMEM((_BB, h), jnp.float32),
            pltpu.SemaphoreType.DMA,
            pltpu.SemaphoreType.DMA,
        ],
        compiler_params=pltpu.CompilerParams(use_tc_tiling_on_sc=False),
    )(idxT, wemb_lin)
    out5 = _tc_epilogue(emb4, posemb, gamma, beta)
    return out5.transpose(2, 4, 0, 1, 3).reshape(batch, seq, h)


# speedup vs baseline: 10.1975x; 10.1975x over previous
import jax
import jax.numpy as jnp
from jax import lax
from jax.experimental import pallas as pl
from jax.experimental.pallas import tpu as pltpu
from jax.experimental.pallas import tpu_sc as plsc

_NC, _NS = 2, 16
_NW = _NC * _NS
_BB = 128
_UNROLL = 8

_DNUMS = lax.GatherDimensionNumbers(offset_dims=(), collapsed_slice_dims=(0,),
                                    start_index_map=(0,))


def _perm(v, mask):
    idx = (lax.iota(jnp.int32, 16) ^ mask).reshape(16, 1)
    return lax.gather(v, idx, _DNUMS, slice_sizes=(1,),
                      mode=lax.GatherScatterMode.PROMISE_IN_BOUNDS)


def _allreduce(v):
    for m in (1, 2, 4, 8):
        v = v + _perm(v, m)
    return v


def _fused_body(idxT_hbm, wemb_hbm, pos_hbm, g_hbm, b_hbm, out_hbm,
                idx_all, rows_a, rows_b, ost_a, ost_b, pos_v, g_v, b_v,
                gsem, osem):
    c = lax.axis_index("c")
    s = lax.axis_index("s")
    w = s * _NC + c
    seq = idxT_hbm.shape[0]
    h = wemb_hbm.shape[1]
    nk = h // 16
    b0 = w * _BB

    pltpu.sync_copy(pos_hbm, pos_v)
    pltpu.sync_copy(g_hbm, g_v)
    pltpu.sync_copy(b_hbm, b_v)
    pltpu.sync_copy(idxT_hbm.at[:, pl.ds(b0, _BB)], idx_all)

    pltpu.async_copy(wemb_hbm.at[idx_all.at[0]], rows_a, gsem)

    def halfstep(t, rows_v, nxt_v, ost_v):
        pltpu.make_async_copy(wemb_hbm.at[idx_all.at[t]], rows_v, gsem).wait()

        @pl.when(t + 1 < seq)
        def _start_next():
            pltpu.async_copy(wemb_hbm.at[idx_all.at[t + 1]], nxt_v, gsem)

        @pl.when(t >= 2)
        def _drain_old_writeback():
            pltpu.make_async_copy(
                ost_v, out_hbm.at[pl.ds(b0, _BB), t - 2], osem).wait()

        p_regs = [pos_v[t, pl.ds(16 * k, 16)] for k in range(nk)]
        g_regs = [g_v[pl.ds(16 * k, 16)] for k in range(nk)]
        b_regs = [b_v[pl.ds(16 * k, 16)] for k in range(nk)]

        def tok_group(j, carry2):
            for jj in range(_UNROLL):
                tk = j * _UNROLL + jj
                e = [rows_v[tk, pl.ds(16 * k, 16)] + p_regs[k]
                     for k in range(nk)]
                u = _allreduce((e[0] + e[1]) + (e[2] + e[3])) * (1.0 / 64.0)
                d = [ek - u for ek in e]
                q = (d[0] * d[0] + d[1] * d[1]) + (d[2] * d[2] + d[3] * d[3])
                vv = _allreduce(q) * (1.0 / 64.0) + 1e-12
                yi = (jnp.full((16,), 0x5F3759DF, dtype=jnp.int32)
                      - (lax.bitcast_convert_type(vv, jnp.int32) >> 1))
                y = lax.bitcast_convert_type(yi, jnp.float32)
                xh = vv * 0.5
                y = y * (1.5 - xh * y * y)
                y = y * (1.5 - xh * y * y)
                for k in range(nk):
                    ost_v[tk, pl.ds(16 * k, 16)] = (
                        d[k] * (y * g_regs[k]) + b_regs[k])
            return carry2

        lax.fori_loop(0, _BB // _UNROLL, tok_group, 0)
        pltpu.async_copy(ost_v, out_hbm.at[pl.ds(b0, _BB), t], osem)

    def pair(i, carry):
        t = i * 2
        halfstep(t, rows_a, rows_b, ost_a)
        halfstep(t + 1, rows_b, rows_a, ost_b)
        return carry

    lax.fori_loop(0, seq // 2, pair, 0)
    pltpu.make_async_copy(ost_a, out_hbm.at[pl.ds(b0, _BB), seq - 2],
                          osem).wait()
    pltpu.make_async_copy(ost_b, out_hbm.at[pl.ds(b0, _BB), seq - 1],
                          osem).wait()


_TCOLS = 32768


def _tbl_body(in_ref, out_ref):
    x = in_ref[...]
    xt = x.T
    n = xt.shape[0]
    out_ref[...] = jnp.concatenate([xt[:n // 2], xt[n // 2:]], axis=1)


def _convert_table(wemb):
    v, h = wemb.shape
    pairs = 128 // h
    grid = (v + _TCOLS - 1) // _TCOLS
    rows = grid * (_TCOLS // pairs)
    out = pl.pallas_call(
        _tbl_body,
        grid=(grid,),
        in_specs=[pl.BlockSpec((h, _TCOLS), lambda i: (0, i))],
        out_specs=pl.BlockSpec((_TCOLS // pairs, 128), lambda i: (i, 0)),
        out_shape=jax.ShapeDtypeStruct((rows, 128), jnp.float32),
    )(wemb.T)
    return out.reshape(rows * pairs, h)


def _remap_indices(v):
    half = _TCOLS // 2
    return (((v // _TCOLS) * _TCOLS) + ((v % half) * 2) + ((v // half) % 2))


def kernel(inputni, wemb, posemb, gamma, beta):
    batch, seq = inputni.shape
    h = wemb.shape[1]
    idxT = _remap_indices(inputni.T.astype(jnp.int32))
    wemb = _convert_table(wemb)
    mesh = plsc.VectorSubcoreMesh(core_axis_name="c", subcore_axis_name="s")
    out = pl.kernel(
        _fused_body,
        out_type=jax.ShapeDtypeStruct((batch, seq, h), jnp.float32),
        mesh=mesh,
        scratch_types=[
            pltpu.VMEM((seq, _BB), jnp.int32),
            pltpu.VMEM((_BB, h), jnp.float32),
            pltpu.VMEM((_BB, h), jnp.float32),
            pltpu.VMEM((_BB, h), jnp.float32),
            pltpu.VMEM((_BB, h), jnp.float32),
            pltpu.VMEM((seq, h), jnp.float32),
            pltpu.VMEM((h,), jnp.float32),
            pltpu.VMEM((h,), jnp.float32),
            pltpu.SemaphoreType.DMA,
            pltpu.SemaphoreType.DMA,
        ],
        compiler_params=pltpu.CompilerParams(use_tc_tiling_on_sc=False),
    )(idxT, wemb, posemb, gamma, beta)
    return out


# speedup vs baseline: 10.8166x; 1.0607x over previous
import jax
import jax.numpy as jnp
from jax import lax
from jax.experimental import pallas as pl
from jax.experimental.pallas import tpu as pltpu
from jax.experimental.pallas import tpu_sc as plsc

_NC, _NS = 2, 16
_NW = _NC * _NS
_BB = 128
_UNROLL = 16

_DNUMS = lax.GatherDimensionNumbers(offset_dims=(), collapsed_slice_dims=(0,),
                                    start_index_map=(0,))


def _perm(v, mask):
    idx = (lax.iota(jnp.int32, 16) ^ mask).reshape(16, 1)
    return lax.gather(v, idx, _DNUMS, slice_sizes=(1,),
                      mode=lax.GatherScatterMode.PROMISE_IN_BOUNDS)


def _allreduce(v):
    for m in (1, 2, 4, 8):
        v = v + _perm(v, m)
    return v


def _fused_body(idxT_hbm, wemb_hbm, pos_hbm, g_hbm, b_hbm, out_hbm,
                idx_all, rows_a, rows_b, ost_a, ost_b, pos_v, g_v, b_v,
                gsem, osem):
    c = lax.axis_index("c")
    s = lax.axis_index("s")
    w = s * _NC + c
    seq = idxT_hbm.shape[0]
    h = wemb_hbm.shape[1]
    nk = h // 16
    b0 = w * _BB

    pltpu.sync_copy(pos_hbm, pos_v)
    pltpu.sync_copy(g_hbm, g_v)
    pltpu.sync_copy(b_hbm, b_v)
    pltpu.sync_copy(idxT_hbm.at[:, pl.ds(b0, _BB)], idx_all)

    pltpu.async_copy(wemb_hbm.at[idx_all.at[0]], rows_a, gsem)

    def halfstep(t, rows_v, nxt_v, ost_v):
        pltpu.make_async_copy(wemb_hbm.at[idx_all.at[t]], rows_v, gsem).wait()

        @pl.when(t + 1 < seq)
        def _start_next():
            pltpu.async_copy(wemb_hbm.at[idx_all.at[t + 1]], nxt_v, gsem)

        @pl.when(t >= 2)
        def _drain_old_writeback():
            pltpu.make_async_copy(
                ost_v, out_hbm.at[pl.ds(b0, _BB), t - 2], osem).wait()

        p_regs = [pos_v[t, pl.ds(16 * k, 16)] for k in range(nk)]
        g_regs = [g_v[pl.ds(16 * k, 16)] for k in range(nk)]
        b_regs = [b_v[pl.ds(16 * k, 16)] for k in range(nk)]

        def tok_group(j, carry2):
            for jj in range(_UNROLL):
                tk = j * _UNROLL + jj
                e = [rows_v[tk, pl.ds(16 * k, 16)] + p_regs[k]
                     for k in range(nk)]
                u = _allreduce((e[0] + e[1]) + (e[2] + e[3])) * (1.0 / 64.0)
                d = [ek - u for ek in e]
                q = (d[0] * d[0] + d[1] * d[1]) + (d[2] * d[2] + d[3] * d[3])
                vv = _allreduce(q) * (1.0 / 64.0) + 1e-12
                yi = (jnp.full((16,), 0x5F3759DF, dtype=jnp.int32)
                      - (lax.bitcast_convert_type(vv, jnp.int32) >> 1))
                y = lax.bitcast_convert_type(yi, jnp.float32)
                xh = vv * 0.5
                y = y * (1.5 - xh * y * y)
                y = y * (1.5 - xh * y * y)
                for k in range(nk):
                    ost_v[tk, pl.ds(16 * k, 16)] = (
                        d[k] * (y * g_regs[k]) + b_regs[k])
            return carry2

        lax.fori_loop(0, _BB // _UNROLL, tok_group, 0)
        pltpu.async_copy(ost_v, out_hbm.at[pl.ds(b0, _BB), t], osem)

    def pair(i, carry):
        t = i * 2
        halfstep(t, rows_a, rows_b, ost_a)
        halfstep(t + 1, rows_b, rows_a, ost_b)
        return carry

    lax.fori_loop(0, seq // 2, pair, 0)
    pltpu.make_async_copy(ost_a, out_hbm.at[pl.ds(b0, _BB), seq - 2],
                          osem).wait()
    pltpu.make_async_copy(ost_b, out_hbm.at[pl.ds(b0, _BB), seq - 1],
                          osem).wait()


_TCOLS = 32768


def _tbl_body(in_ref, out_ref):
    x = in_ref[...]
    xt = x.T
    n = xt.shape[0]
    out_ref[...] = jnp.concatenate([xt[:n // 2], xt[n // 2:]], axis=1)


def _convert_table(wemb):
    v, h = wemb.shape
    pairs = 128 // h
    grid = (v + _TCOLS - 1) // _TCOLS
    rows = grid * (_TCOLS // pairs)
    out = pl.pallas_call(
        _tbl_body,
        grid=(grid,),
        in_specs=[pl.BlockSpec((h, _TCOLS), lambda i: (0, i))],
        out_specs=pl.BlockSpec((_TCOLS // pairs, 128), lambda i: (i, 0)),
        out_shape=jax.ShapeDtypeStruct((rows, 128), jnp.float32),
    )(wemb.T)
    return out.reshape(rows * pairs, h)


def _remap_indices(v):
    half = _TCOLS // 2
    return (((v // _TCOLS) * _TCOLS) + ((v % half) * 2) + ((v // half) % 2))


def kernel(inputni, wemb, posemb, gamma, beta):
    batch, seq = inputni.shape
    h = wemb.shape[1]
    idxT = _remap_indices(inputni.T.astype(jnp.int32))
    wemb = _convert_table(wemb)
    mesh = plsc.VectorSubcoreMesh(core_axis_name="c", subcore_axis_name="s")
    out = pl.kernel(
        _fused_body,
        out_type=jax.ShapeDtypeStruct((batch, seq, h), jnp.float32),
        mesh=mesh,
        scratch_types=[
            pltpu.VMEM((seq, _BB), jnp.int32),
            pltpu.VMEM((_BB, h), jnp.float32),
            pltpu.VMEM((_BB, h), jnp.float32),
            pltpu.VMEM((_BB, h), jnp.float32),
            pltpu.VMEM((_BB, h), jnp.float32),
            pltpu.VMEM((seq, h), jnp.float32),
            pltpu.VMEM((h,), jnp.float32),
            pltpu.VMEM((h,), jnp.float32),
            pltpu.SemaphoreType.DMA,
            pltpu.SemaphoreType.DMA,
        ],
        compiler_params=pltpu.CompilerParams(use_tc_tiling_on_sc=False),
    )(idxT, wemb, posemb, gamma, beta)
    return out
